# baseline (speedup 1.0000x reference)
import functools
import math

import jax
import jax.numpy as jnp
from jax import lax
from jax.experimental import pallas as pl
from jax.experimental.pallas import tpu as pltpu
from jax.experimental.pallas import tpu_sc as plsc

NC = 2
NS = 16
NW = NC * NS
L = 16
SUB = 128


@functools.lru_cache(maxsize=None)
def _build(B, V, D, C):
    assert B % (NW * C) == 0 and C % SUB == 0 and D % L == 0
    b_per_w = B // NW
    n_chunks = b_per_w // C
    scale = float(math.sqrt(D))
    mesh = plsc.VectorSubcoreMesh(
        core_axis_name="c", subcore_axis_name="s",
        num_cores=NC, num_subcores=NS)

    @functools.partial(
        pl.kernel,
        out_type=jax.ShapeDtypeStruct((B, D), jnp.float32),
        mesh=mesh,
        scratch_types=[
            pltpu.VMEM((C,), jnp.int32),
            pltpu.VMEM((C, D), jnp.float32),
            pltpu.SemaphoreType.DMA,
        ],
        compiler_params=pltpu.CompilerParams(use_tc_tiling_on_sc=False),
    )
    def emb(tokens_hbm, table_hbm, out_hbm, idx_v, rows_v, sem):
        wid = lax.axis_index("s") * NC + lax.axis_index("c")
        base = wid * b_per_w

        def chunk(g, carry):
            off = base + g * C
            pltpu.sync_copy(tokens_hbm.at[pl.ds(off, C)], idx_v)
            copies = [
                pltpu.async_copy(
                    table_hbm.at[idx_v.at[pl.ds(s * SUB, SUB)]],
                    rows_v.at[pl.ds(s * SUB, SUB)],
                    sem,
                )
                for s in range(C // SUB)
            ]
            for c in copies:
                c.wait()

            def scale_row(i, carry):
                for j in range(D // L):
                    sl = pl.ds(j * L, L)
                    rows_v[i, sl] = rows_v[i, sl] * scale
                return carry

            lax.fori_loop(0, C, scale_row, 0, unroll=2)
            pltpu.sync_copy(rows_v, out_hbm.at[pl.ds(off, C)])
            return carry

        lax.fori_loop(0, n_chunks, chunk, 0)

    return emb


@jax.jit
def kernel(tokens, table):
    B = tokens.shape[0] * tokens.shape[1]
    V, D = table.shape
    idx = tokens.reshape(B).astype(jnp.int32)
    out = _build(B, V, D, 512)(idx, table)
    return out.reshape(*tokens.shape, D)


# speedup vs baseline: 1.0935x; 1.0935x over previous
import functools
import math

import jax
import jax.numpy as jnp
from jax import lax
from jax.experimental import pallas as pl
from jax.experimental.pallas import tpu as pltpu
from jax.experimental.pallas import tpu_sc as plsc

NC = 2
NS = 16
NW = NC * NS
L = 16
NBUF = 4


@functools.lru_cache(maxsize=None)
def _build(B, V, D, C):
    assert B % (NW * C) == 0 and D % L == 0 and C % 8 == 0
    b_per_w = B // NW
    n_chunks = b_per_w // C
    assert n_chunks % NBUF == 0 and n_chunks >= 2 * NBUF
    scale = float(math.sqrt(D))
    mesh = plsc.VectorSubcoreMesh(
        core_axis_name="c", subcore_axis_name="s",
        num_cores=NC, num_subcores=NS)

    @functools.partial(
        pl.kernel,
        out_type=jax.ShapeDtypeStruct((B, D), jnp.float32),
        mesh=mesh,
        scratch_types=[
            [pltpu.VMEM((C,), jnp.int32) for _ in range(NBUF)],
            [pltpu.VMEM((C, D), jnp.float32) for _ in range(NBUF)],
            [pltpu.SemaphoreType.DMA for _ in range(NBUF)],
            [pltpu.SemaphoreType.DMA for _ in range(NBUF)],
        ],
        compiler_params=pltpu.CompilerParams(use_tc_tiling_on_sc=False),
    )
    def emb(tokens_hbm, table_hbm, out_hbm, idx_bufs, row_bufs, gsems, osems):
        wid = lax.axis_index("s") * NC + lax.axis_index("c")
        base = wid * b_per_w

        def issue_gather(g, b):
            off = base + g * C
            pltpu.sync_copy(tokens_hbm.at[pl.ds(off, C)], idx_bufs[b])
            pltpu.async_copy(table_hbm.at[idx_bufs[b]], row_bufs[b], gsems[b])

        def wait_gather(b):
            pltpu.make_async_copy(
                table_hbm.at[idx_bufs[b]], row_bufs[b], gsems[b]).wait()

        def out_copy(g, b):
            return pltpu.make_async_copy(
                row_bufs[b], out_hbm.at[pl.ds(base + g * C, C)], osems[b])

        for g in range(NBUF - 1):
            issue_gather(g, g)

        def step(t, carry):
            for b in range(NBUF):
                g = t * NBUF + b
                wait_gather(b)

                @plsc.parallel_loop(0, C, 1, unroll=4)
                def _scale(i):
                    for j in range(D // L):
                        sl = pl.ds(j * L, L)
                        row_bufs[b][i, sl] = row_bufs[b][i, sl] * scale

                out_copy(g, b).start()
                nb = (b + NBUF - 1) % NBUF
                g_next = g + NBUF - 1

                @pl.when(g_next < n_chunks)
                def _prefetch():
                    @pl.when(g >= 1)
                    def _drain_prev_out():
                        out_copy(g - 1, nb).wait()
                    issue_gather(g_next, nb)

            return carry

        lax.fori_loop(0, n_chunks // NBUF, step, 0)
        for b in range(NBUF):
            out_copy(n_chunks - NBUF + b, b).wait()

    return emb


@jax.jit
def kernel(tokens, table):
    B = tokens.shape[0] * tokens.shape[1]
    V, D = table.shape
    idx = tokens.reshape(B).astype(jnp.int32)
    out = _build(B, V, D, 400)(idx, table)
    return out.reshape(*tokens.shape, D)
